# Initial kernel scaffold; baseline (speedup 1.0000x reference)
import jax
import jax.numpy as jnp
from jax.experimental import pallas as pl


def kernel(emb_id, emb_name, emb_gics, item_id_idx, item_name_tokens, item_gics_idx):
    raise NotImplementedError("write your pallas kernel here")


# speedup vs baseline: 3.7906x; 3.7906x over previous
import jax
import jax.numpy as jnp
from jax import lax
from jax.experimental import pallas as pl
from jax.experimental.pallas import tpu as pltpu
from jax.experimental.pallas import tpu_sc as plsc

B = 4096
NAME_LEN = 20
D_ID = 8
D_NAME = 16
D_GICS = 8
D_OUT = D_ID + D_NAME + D_GICS

NC, NS, L = 2, 16, 16
NW = NC * NS
BPW = B // NW
NGRP = BPW // L
TOKS = BPW * NAME_LEN
CHUNK = 128
NCHUNK = TOKS // CHUNK


def _body(emb_id, emb_name, emb_gics, idv, tok3, gicsv, out_hbm,
          ididx_v, gicsidx_v, tok_v, idrows_v, gicsrows_v, namerows_v,
          emb0_v, emb0T_v, z_v, rcnt_v, out_v, sem):
    wid = lax.axis_index("c") * NS + lax.axis_index("s")
    base = wid * BPW
    iota = lax.iota(jnp.int32, L)

    pltpu.sync_copy(idv.at[pl.ds(base, BPW)], ididx_v)
    pltpu.sync_copy(gicsv.at[pl.ds(base, BPW)], gicsidx_v)
    pltpu.sync_copy(tok3.at[wid], tok_v)
    pltpu.sync_copy(emb_name.at[pl.ds(0, 1)], emb0_v)

    copies = [
        pltpu.async_copy(emb_id.at[ididx_v], idrows_v, sem),
        pltpu.async_copy(emb_gics.at[gicsidx_v], gicsrows_v, sem),
    ]
    for c in range(NCHUNK):
        copies.append(pltpu.async_copy(
            emb_name.at[tok_v.at[c]],
            namerows_v.at[pl.ds(c * CHUNK, CHUNK)], sem))

    e0vec = emb0_v[0, :]
    for c in range(L):
        plsc.store_scatter(emb0T_v, [iota, jnp.full((L,), c, jnp.int32)],
                           e0vec)

    iota20 = iota * NAME_LEN

    def count_group(g, carry):
        cnt = jnp.zeros((L,), jnp.float32)
        for l in range(NAME_LEN):
            flat = iota20 + (g * (L * NAME_LEN) + l)
            t = plsc.load_gather(tok_v, [flat >> 7, flat & (CHUNK - 1)])
            cnt = cnt + jnp.where(t != 0, 1.0, 0.0)
        z_v[pl.ds(g * L, L)] = jnp.float32(NAME_LEN) - cnt
        rcnt_v[pl.ds(g * L, L)] = 1.0 / jnp.maximum(cnt, 1.0)
        return carry

    lax.fori_loop(0, NGRP, count_group, 0)

    for cp in copies:
        cp.wait()

    hi8 = iota >> 3
    col8 = iota & 7

    def pair(j, carry):
        row = 2 * j + hi8
        plsc.store_scatter(out_v, [row, col8],
                           plsc.load_gather(idrows_v, [row, col8]))
        plsc.store_scatter(out_v, [row, col8 + (D_ID + D_NAME)],
                           plsc.load_gather(gicsrows_v, [row, col8]))
        return carry

    lax.fori_loop(0, BPW // 2, pair, 0)

    def group(g, carry):
        rows = [iota20 + (g * (L * NAME_LEN) + l) for l in range(NAME_LEN)]
        items = iota + g * L
        z = z_v[pl.ds(g * L, L)]
        rc = rcnt_v[pl.ds(g * L, L)]
        for d in range(D_NAME):
            cold = jnp.full((L,), d, jnp.int32)
            acc = plsc.load_gather(namerows_v, [rows[0], cold])
            for l in range(1, NAME_LEN):
                acc = acc + plsc.load_gather(namerows_v, [rows[l], cold])
            e0 = emb0T_v[d, :]
            val = (acc - z * e0) * rc
            plsc.store_scatter(
                out_v, [items, jnp.full((L,), D_ID + d, jnp.int32)], val)
        return carry

    lax.fori_loop(0, NGRP, group, 0)

    pltpu.sync_copy(out_v, out_hbm.at[pl.ds(base, BPW)])


def kernel(emb_id, emb_name, emb_gics, item_id_idx, item_name_tokens,
           item_gics_idx):
    idv = item_id_idx.astype(jnp.int32)
    gicsv = item_gics_idx.astype(jnp.int32)
    tok3 = item_name_tokens.astype(jnp.int32).reshape(NW, NCHUNK, CHUNK)
    mesh = plsc.VectorSubcoreMesh(core_axis_name="c", subcore_axis_name="s")
    f = pl.kernel(
        _body,
        out_type=jax.ShapeDtypeStruct((B, D_OUT), jnp.float32),
        mesh=mesh,
        compiler_params=pltpu.CompilerParams(
            needs_layout_passes=False, use_tc_tiling_on_sc=False),
        scratch_types=[
            pltpu.VMEM((BPW,), jnp.int32),
            pltpu.VMEM((BPW,), jnp.int32),
            pltpu.VMEM((NCHUNK, CHUNK), jnp.int32),
            pltpu.VMEM((BPW, D_ID), jnp.float32),
            pltpu.VMEM((BPW, D_GICS), jnp.float32),
            pltpu.VMEM((TOKS, D_NAME), jnp.float32),
            pltpu.VMEM((1, D_NAME), jnp.float32),
            pltpu.VMEM((L, L), jnp.float32),
            pltpu.VMEM((BPW,), jnp.float32),
            pltpu.VMEM((BPW,), jnp.float32),
            pltpu.VMEM((BPW, D_OUT), jnp.float32),
            pltpu.SemaphoreType.DMA,
        ],
    )
    return f(emb_id, emb_name, emb_gics, idv, tok3, gicsv)


# speedup vs baseline: 6.6057x; 1.7427x over previous
import jax
import jax.numpy as jnp
from jax import lax
from jax.experimental import pallas as pl
from jax.experimental.pallas import tpu as pltpu
from jax.experimental.pallas import tpu_sc as plsc

B = 4096
NAME_LEN = 20
D_ID = 8
D_NAME = 16
D_GICS = 8
D_OUT = D_ID + D_NAME + D_GICS
ID_VOCAB = 100001
GICS_VOCAB = 1001

NC, NS, L = 2, 16, 16
NW = NC * NS
BPW = B // NW
NGRP = BPW // L


def _treesum(vs):
    while len(vs) > 1:
        vs = [a + b for a, b in zip(vs[::2], vs[1::2])] + (
            [vs[-1]] if len(vs) % 2 else [])
    return vs[0]


def _body(emb_id_f, emb_name, emb_gics_f, idv, tok_t, gicsv, out_hbm,
          ididx_v, gicsidx_v, idix_v, gicsix_v, tok_v, namerows_v,
          emb0_v, emb0T_v, z_v, rcnt_v, out_tv, sem):
    wid = lax.axis_index("c") * NS + lax.axis_index("s")
    base = wid * BPW
    iota = lax.iota(jnp.int32, L)

    pltpu.sync_copy(idv.at[pl.ds(base, BPW)], ididx_v)
    pltpu.sync_copy(gicsv.at[pl.ds(base, BPW)], gicsidx_v)
    for l in range(NAME_LEN):
        pltpu.sync_copy(tok_t.at[l, pl.ds(base, BPW)], tok_v.at[l])
    pltpu.sync_copy(emb_name.at[pl.ds(0, 1)], emb0_v)

    for d in range(D_ID):
        for c in range(NGRP):
            sl = pl.ds(c * L, L)
            idix_v[d, sl] = ididx_v[sl] + d * ID_VOCAB
            gicsix_v[d, sl] = gicsidx_v[sl] + d * GICS_VOCAB

    copies = []
    for d in range(D_ID):
        copies.append(pltpu.async_copy(
            emb_id_f.at[idix_v.at[d]], out_tv.at[d], sem))
        copies.append(pltpu.async_copy(
            emb_gics_f.at[gicsix_v.at[d]],
            out_tv.at[D_ID + D_NAME + d], sem))
    for l in range(NAME_LEN):
        copies.append(pltpu.async_copy(
            emb_name.at[tok_v.at[l]],
            namerows_v.at[pl.ds(l * BPW, BPW)], sem))

    e0vec = emb0_v[0, :]
    for c in range(L):
        plsc.store_scatter(emb0T_v, [iota, jnp.full((L,), c, jnp.int32)],
                           e0vec)

    def count_group(g, carry):
        sl = pl.ds(g * L, L)
        ones = [jnp.where(tok_v[l, sl] != 0, 1.0, 0.0)
                for l in range(NAME_LEN)]
        cnt = _treesum(ones)
        z_v[sl] = jnp.float32(NAME_LEN) - cnt
        rcnt_v[sl] = 1.0 / jnp.maximum(cnt, 1.0)
        return carry

    lax.fori_loop(0, NGRP, count_group, 0)

    for cp in copies:
        cp.wait()

    def group(g, carry):
        sl = pl.ds(g * L, L)
        rows = [iota + (l * BPW + g * L) for l in range(NAME_LEN)]
        z = z_v[sl]
        rc = rcnt_v[sl]
        for d in range(D_NAME):
            cold = jnp.full((L,), d, jnp.int32)
            acc = _treesum([plsc.load_gather(namerows_v, [rows[l], cold])
                            for l in range(NAME_LEN)])
            out_tv[D_ID + d, sl] = (acc - z * emb0T_v[d, :]) * rc
        return carry

    lax.fori_loop(0, NGRP, group, 0)

    for r in range(D_OUT):
        pltpu.sync_copy(out_tv.at[r], out_hbm.at[r, pl.ds(base, BPW)])


def kernel(emb_id, emb_name, emb_gics, item_id_idx, item_name_tokens,
           item_gics_idx):
    idv = item_id_idx.astype(jnp.int32)
    gicsv = item_gics_idx.astype(jnp.int32)
    tok_t = item_name_tokens.astype(jnp.int32).T
    emb_id_f = emb_id.T.reshape(-1)
    emb_gics_f = emb_gics.T.reshape(-1)
    mesh = plsc.VectorSubcoreMesh(core_axis_name="c", subcore_axis_name="s")
    f = pl.kernel(
        _body,
        out_type=jax.ShapeDtypeStruct((D_OUT, B), jnp.float32),
        mesh=mesh,
        compiler_params=pltpu.CompilerParams(
            needs_layout_passes=False, use_tc_tiling_on_sc=False),
        scratch_types=[
            pltpu.VMEM((BPW,), jnp.int32),
            pltpu.VMEM((BPW,), jnp.int32),
            pltpu.VMEM((D_ID, BPW), jnp.int32),
            pltpu.VMEM((D_GICS, BPW), jnp.int32),
            pltpu.VMEM((NAME_LEN, BPW), jnp.int32),
            pltpu.VMEM((NAME_LEN * BPW, D_NAME), jnp.float32),
            pltpu.VMEM((1, D_NAME), jnp.float32),
            pltpu.VMEM((L, L), jnp.float32),
            pltpu.VMEM((BPW,), jnp.float32),
            pltpu.VMEM((BPW,), jnp.float32),
            pltpu.VMEM((D_OUT, BPW), jnp.float32),
            pltpu.SemaphoreType.DMA,
        ],
    )
    return f(emb_id_f, emb_name, emb_gics_f, idv, tok_t, gicsv).T


# speedup vs baseline: 7.7711x; 1.1764x over previous
import jax
import jax.numpy as jnp
from jax import lax
from jax.experimental import pallas as pl
from jax.experimental.pallas import tpu as pltpu
from jax.experimental.pallas import tpu_sc as plsc

B = 4096
NAME_LEN = 20
D_ID = 8
D_NAME = 16
D_GICS = 8
D_OUT = D_ID + D_NAME + D_GICS
ID_VOCAB = 100001
GICS_VOCAB = 1001

NC, NS, L = 2, 16, 16
NW = NC * NS
BPW = B // NW
NGRP = BPW // L


def _treesum(vs):
    while len(vs) > 1:
        vs = [a + b for a, b in zip(vs[::2], vs[1::2])] + (
            [vs[-1]] if len(vs) % 2 else [])
    return vs[0]


def _body(emb_id_f, emb_name, emb_gics_f, idv, tok_t, gicsv, out_hbm,
          ididx_v, gicsidx_v, idix_v, gicsix_v, tok_v, namerows_v,
          emb0_v, emb0T_v, z_v, rcnt_v, out_tv, sem):
    wid = lax.axis_index("c") * NS + lax.axis_index("s")
    base = wid * BPW
    iota = lax.iota(jnp.int32, L)

    pltpu.sync_copy(idv.at[pl.ds(base, BPW)], ididx_v)
    pltpu.sync_copy(gicsv.at[pl.ds(base, BPW)], gicsidx_v)
    pltpu.sync_copy(tok_t.at[:, pl.ds(base, BPW)], tok_v)
    pltpu.sync_copy(emb_name.at[pl.ds(0, 1)], emb0_v)

    for d in range(D_ID):
        for c in range(NGRP):
            sl = pl.ds(c * L, L)
            idix_v[d, sl] = ididx_v[sl] + d * ID_VOCAB
            gicsix_v[d, sl] = gicsidx_v[sl] + d * GICS_VOCAB

    copies = []
    for d in range(D_ID):
        copies.append(pltpu.async_copy(
            emb_id_f.at[idix_v.at[d]], out_tv.at[d], sem))
        copies.append(pltpu.async_copy(
            emb_gics_f.at[gicsix_v.at[d]],
            out_tv.at[D_ID + D_NAME + d], sem))
    for l in range(NAME_LEN):
        copies.append(pltpu.async_copy(
            emb_name.at[tok_v.at[l]],
            namerows_v.at[pl.ds(l * BPW, BPW)], sem))

    e0vec = emb0_v[0, :]
    for c in range(L):
        plsc.store_scatter(emb0T_v, [iota, jnp.full((L,), c, jnp.int32)],
                           e0vec)

    def count_group(g, carry):
        sl = pl.ds(g * L, L)
        ones = [jnp.where(tok_v[l, sl] != 0, 1.0, 0.0)
                for l in range(NAME_LEN)]
        cnt = _treesum(ones)
        z_v[sl] = jnp.float32(NAME_LEN) - cnt
        rcnt_v[sl] = 1.0 / jnp.maximum(cnt, 1.0)
        return carry

    lax.fori_loop(0, NGRP, count_group, 0)

    for cp in copies:
        cp.wait()

    def group(g, carry):
        sl = pl.ds(g * L, L)
        rows = [iota + (l * BPW + g * L) for l in range(NAME_LEN)]
        z = z_v[sl]
        rc = rcnt_v[sl]
        for d in range(D_NAME):
            cold = jnp.full((L,), d, jnp.int32)
            acc = _treesum([plsc.load_gather(namerows_v, [rows[l], cold])
                            for l in range(NAME_LEN)])
            out_tv[D_ID + d, sl] = (acc - z * emb0T_v[d, :]) * rc
        return carry

    lax.fori_loop(0, NGRP, group, 0)

    pltpu.sync_copy(out_tv, out_hbm.at[:, pl.ds(base, BPW)])


def kernel(emb_id, emb_name, emb_gics, item_id_idx, item_name_tokens,
           item_gics_idx):
    idv = item_id_idx.astype(jnp.int32)
    gicsv = item_gics_idx.astype(jnp.int32)
    tok_t = item_name_tokens.astype(jnp.int32).T
    emb_id_f = emb_id.T.reshape(-1)
    emb_gics_f = emb_gics.T.reshape(-1)
    mesh = plsc.VectorSubcoreMesh(core_axis_name="c", subcore_axis_name="s")
    f = pl.kernel(
        _body,
        out_type=jax.ShapeDtypeStruct((D_OUT, B), jnp.float32),
        mesh=mesh,
        compiler_params=pltpu.CompilerParams(
            needs_layout_passes=False, use_tc_tiling_on_sc=False),
        scratch_types=[
            pltpu.VMEM((BPW,), jnp.int32),
            pltpu.VMEM((BPW,), jnp.int32),
            pltpu.VMEM((D_ID, BPW), jnp.int32),
            pltpu.VMEM((D_GICS, BPW), jnp.int32),
            pltpu.VMEM((NAME_LEN, BPW), jnp.int32),
            pltpu.VMEM((NAME_LEN * BPW, D_NAME), jnp.float32),
            pltpu.VMEM((1, D_NAME), jnp.float32),
            pltpu.VMEM((L, L), jnp.float32),
            pltpu.VMEM((BPW,), jnp.float32),
            pltpu.VMEM((BPW,), jnp.float32),
            pltpu.VMEM((D_OUT, BPW), jnp.float32),
            pltpu.SemaphoreType.DMA,
        ],
    )
    return f(emb_id_f, emb_name, emb_gics_f, idv, tok_t, gicsv).T
